# baseline (speedup 1.0000x reference)
import jax
import jax.numpy as jnp
import numpy as np
from jax import lax
from jax.experimental import pallas as pl
from jax.experimental.pallas import tpu as pltpu
from jax.experimental.pallas import tpu_sc as plsc

N = 10000
E = 160000
D = 256
DE = 16
L = 2
DH = D // 2

NC = 2
NS = 16
NW = NC * NS
RPT = 320
NPAD = NW * RPT
JUNK = RPT
DEGBASE = RPT + 8
AROWS = DEGBASE + 24
K = 64


def _ln(x, g, b):
    mu = jnp.mean(x, axis=-1, keepdims=True)
    v = jnp.mean((x - mu) ** 2, axis=-1, keepdims=True)
    return g * (x - mu) / jnp.sqrt(v + 1e-5) + b


def _gelu(x):
    return 0.5 * x * (1.0 + lax.erf(x * 0.7071067811865476))


def _in_proj_body(x_ref, w_ref, b_ref, g_ref, be_ref, o_ref):
    y = jnp.dot(x_ref[...], w_ref[...], preferred_element_type=jnp.float32)
    y = _gelu(y + b_ref[...])
    o_ref[...] = _ln(y, g_ref[...], be_ref[...])


def _matmul_body(x_ref, w_ref, o_ref):
    o_ref[...] = jnp.dot(x_ref[...], w_ref[...],
                         preferred_element_type=jnp.float32)


GSCALE = 262143.0


def _gate_body(ea_ref, dvv_ref, div_ref, A_ref, B_ref, C_ref, bg1_ref,
               w2_ref, bg2_ref, o_ref):
    ea = ea_ref[...]
    ts = ea[:, 0:1]
    ty = ea[:, 1:2]
    dv = div_ref[...]
    s = jnp.sin(ts * dv)
    c = jnp.cos(ty * dv)
    h = jnp.dot(ea, A_ref[...], preferred_element_type=jnp.float32)
    h += jnp.dot(s, B_ref[...], preferred_element_type=jnp.float32)
    h += jnp.dot(c, C_ref[...], preferred_element_type=jnp.float32)
    h = _gelu(h + bg1_ref[...])
    hw = h * w2_ref[...]
    g0 = jax.nn.sigmoid(jnp.sum(hw[:, :D], axis=-1) + bg2_ref[0, 0])
    g1 = jax.nn.sigmoid(jnp.sum(hw[:, D:], axis=-1) + bg2_ref[0, 1])
    dvv = dvv_ref[0, :]
    q0 = jnp.round(g0 * GSCALE).astype(jnp.int32)
    q1 = jnp.round(g1 * GSCALE).astype(jnp.int32)
    o_ref[0, :] = dvv | (q0 << 14)
    o_ref[1, :] = dvv | (q1 << 14)


def _fuse_body(agg_ref, deg_ref, h_ref, wd_ref, bd_ref, gl_ref, bl_ref,
               gr_ref, br_ref, o_ref):
    h = h_ref[...]
    deg = jnp.maximum(deg_ref[...][:, 0:1], 1.0)
    out = agg_ref[...] / deg
    out += jnp.dot(h, wd_ref[...], preferred_element_type=jnp.float32)
    out += bd_ref[...]
    msg = _gelu(_ln(out, gl_ref[...], bl_ref[...]))
    o_ref[...] = _ln(msg + h, gr_ref[...], br_ref[...])


_ROWB = 1000


def _in_proj(x, w, b, g, be):
    xs = pl.BlockSpec((_ROWB, D), lambda i: (i, 0))
    wspec = pl.BlockSpec((D, D), lambda i: (0, 0))
    vspec = pl.BlockSpec((1, D), lambda i: (0, 0))
    return pl.pallas_call(
        _in_proj_body,
        grid=(N // _ROWB,),
        in_specs=[xs, wspec, vspec, vspec, vspec],
        out_specs=xs,
        out_shape=jax.ShapeDtypeStruct((N, D), jnp.float32),
    )(x, w, b.reshape(1, D), g.reshape(1, D), be.reshape(1, D))


def _matmul(x, w):
    xs = pl.BlockSpec((_ROWB, D), lambda i: (i, 0))
    wspec = pl.BlockSpec((D, D), lambda i: (0, 0))
    return pl.pallas_call(
        _matmul_body,
        grid=(N // _ROWB,),
        in_specs=[xs, wspec],
        out_specs=xs,
        out_shape=jax.ShapeDtypeStruct((N, D), jnp.float32),
    )(x, w)


_EB = 2048


def _gate(ea, dvv, Wg1, bg1, Wg2, bg2):
    A = jnp.concatenate([Wg1[0, :DE, :], Wg1[1, :DE, :]], axis=1)
    B = jnp.concatenate([Wg1[0, DE + 0::2, :], Wg1[1, DE + 0::2, :]], axis=1)
    C = jnp.concatenate([Wg1[0, DE + 1::2, :], Wg1[1, DE + 1::2, :]], axis=1)
    bg1c = jnp.concatenate([bg1[0], bg1[1]]).reshape(1, 2 * D)
    w2 = jnp.concatenate([Wg2[0, :, 0], Wg2[1, :, 0]]).reshape(1, 2 * D)
    bg2c = bg2.reshape(1, L)
    div = jnp.exp(jnp.arange(DH, dtype=jnp.float32)
                  * (-(np.log(10000.0) / DH))).reshape(1, DH)

    easpec = pl.BlockSpec((_EB, DE), lambda i: (i, 0))
    dvspec = pl.BlockSpec((1, _EB), lambda i: (0, i))
    ospec = pl.BlockSpec((L, _EB), lambda i: (0, i))
    fulls = [
        pl.BlockSpec((1, DH), lambda i: (0, 0)),
        pl.BlockSpec((DE, 2 * D), lambda i: (0, 0)),
        pl.BlockSpec((DH, 2 * D), lambda i: (0, 0)),
        pl.BlockSpec((DH, 2 * D), lambda i: (0, 0)),
        pl.BlockSpec((1, 2 * D), lambda i: (0, 0)),
        pl.BlockSpec((1, 2 * D), lambda i: (0, 0)),
        pl.BlockSpec((1, L), lambda i: (0, 0)),
    ]
    return pl.pallas_call(
        _gate_body,
        grid=(pl.cdiv(E, _EB),),
        in_specs=[easpec, dvspec] + fulls,
        out_specs=ospec,
        out_shape=jax.ShapeDtypeStruct((L, E), jnp.int32),
    )(ea, dvv.reshape(1, E), div, A, B, C, bg1c, w2, bg2c)


def _fuse(agg, deg, h, wd, bd, gl, bl, gr, br):
    xs = pl.BlockSpec((_ROWB, D), lambda i: (i, 0))
    degspec = pl.BlockSpec((_ROWB, 1), lambda i: (i, 0))
    wspec = pl.BlockSpec((D, D), lambda i: (0, 0))
    vspec = pl.BlockSpec((1, D), lambda i: (0, 0))
    return pl.pallas_call(
        _fuse_body,
        grid=(N // _ROWB,),
        in_specs=[xs, degspec, xs, wspec, vspec, vspec, vspec, vspec, vspec],
        out_specs=xs,
        out_shape=jax.ShapeDtypeStruct((N, D), jnp.float32),
    )(agg, deg, h, wd, bd.reshape(1, D), gl.reshape(1, D), bl.reshape(1, D),
      gr.reshape(1, D), br.reshape(1, D))


QW = E + 16
REG = 2 * QW + 16
BSUB = 1280
NBSUB = E // BSUB
GPB = 8
NBATCH = BSUB // (16 * GPB)


def _sc_build_body(sv_hbm, dv_hbm, q_hbm,
                   svt, dvt, pad, idxb, idx2b, svb, peb, sem):
    c = lax.axis_index("c")
    s = lax.axis_index("s")
    w = c * NS + s
    base = w * RPT
    rbase = w * REG
    iota16 = lax.iota(jnp.int32, 16)
    pad[pl.ds(0, 16)] = jnp.zeros((16,), jnp.float32)

    def block(o, q):
        eo = o * BSUB
        pltpu.sync_copy(sv_hbm.at[pl.ds(eo, BSUB)], svt)
        pltpu.sync_copy(dv_hbm.at[pl.ds(eo, BSUB)], dvt)

        def batch(bi, q):
            for t in range(GPB):
                j = bi * GPB + t
                esl = pl.ds(j * 16, 16)
                loc = (dvt[esl] & 16383) - base
                m = (loc >= 0) & (loc < RPT)
                mi = jnp.where(m, jnp.float32(1), jnp.float32(0))
                p = mi
                for k in (1, 2, 4, 8):
                    pad[pl.ds(16, 16)] = p
                    p = p + pad[pl.ds(16 - k, 16)]
                excl = (p - mi).astype(jnp.int32)
                pos = jnp.where(m, q + excl, E)
                b = t * 16
                idxb[pl.ds(b, 16)] = rbase + pos
                idx2b[pl.ds(b, 16)] = rbase + QW + pos
                svb[pl.ds(b, 16)] = svt[esl]
                peb[pl.ds(b, 16)] = (eo + j * 16 + iota16) | (loc << 18)
                q = q + p[15].astype(jnp.int32)
            pltpu.async_copy(svb, q_hbm.at[idxb], sem).wait()
            pltpu.async_copy(peb, q_hbm.at[idx2b], sem).wait()
            return q

        return lax.fori_loop(0, NBATCH, batch, q)

    q = lax.fori_loop(0, NBSUB, block, jnp.int32(0))

    svb[pl.ds(0, 16)] = jnp.zeros((16,), jnp.int32) + q
    pltpu.sync_copy(svb.at[pl.ds(0, 16)],
                    q_hbm.at[pl.ds(rbase + 2 * QW, 16)])


def _sc_build(sv, dv):
    mesh = plsc.VectorSubcoreMesh(core_axis_name="c", subcore_axis_name="s",
                                  num_cores=NC, num_subcores=NS)
    scratch = [
        pltpu.VMEM((BSUB,), jnp.int32),
        pltpu.VMEM((BSUB,), jnp.int32),
        pltpu.VMEM((48,), jnp.float32),
        pltpu.VMEM((16 * GPB,), jnp.int32),
        pltpu.VMEM((16 * GPB,), jnp.int32),
        pltpu.VMEM((16 * GPB,), jnp.int32),
        pltpu.VMEM((16 * GPB,), jnp.int32),
        pltpu.SemaphoreType.DMA,
    ]
    fn = pl.kernel(_sc_build_body,
                   out_type=jax.ShapeDtypeStruct((NW * REG,), jnp.int32),
                   mesh=mesh, scratch_types=scratch)
    return fn(sv, dv)


def _sc_consume_gated(z_hbm, q_hbm, cv_hbm, out_hbm,
                      acc, cntv, svc, pec, eidb, cvc, rows, sem):
    _sc_consume_common(z_hbm, q_hbm, cv_hbm, out_hbm, acc, cntv, svc, pec,
                       eidb, cvc, rows, sem, gated=True)


def _sc_consume_plain(z_hbm, q_hbm, out_hbm,
                      acc, cntv, svc, pec, eidb, cvc, rows, sem):
    _sc_consume_common(z_hbm, q_hbm, None, out_hbm, acc, cntv, svc, pec,
                       eidb, cvc, rows, sem, gated=False)


def _sc_consume_common(z_hbm, q_hbm, cv_hbm, out_hbm,
                       acc, cntv, svc, pec, eidb, cvc, rows, sem, *, gated):
    c = lax.axis_index("c")
    s = lax.axis_index("s")
    w = c * NS + s
    rbase = w * REG
    iota16 = lax.iota(jnp.int32, 16)
    z16 = jnp.zeros((16,), jnp.float32)

    pltpu.sync_copy(q_hbm.at[pl.ds(rbase + 2 * QW, 16)], cntv)
    cnt = cntv[pl.ds(0, 16)][0]

    def _zero(i, _):
        for jj in range(D // 16):
            acc[i, pl.ds(jj * 16, 16)] = z16
        return 0
    lax.fori_loop(0, AROWS, _zero, 0)

    nch = (cnt + (K - 1)) // K

    def chunk(i, _):
        qb = i * K
        pltpu.sync_copy(q_hbm.at[pl.ds(rbase + qb, K)], svc)
        pltpu.sync_copy(q_hbm.at[pl.ds(rbase + QW + qb, K)], pec)
        for t in range(K // 16):
            tsl = pl.ds(t * 16, 16)
            valid = (qb + t * 16 + iota16) < cnt
            svc[tsl] = jnp.where(valid, svc[tsl], 0)
            pe = jnp.where(valid, pec[tsl], JUNK << 18)
            pec[tsl] = pe
            if gated:
                eidb[tsl] = pe & 262143
        if gated:
            pltpu.async_copy(cv_hbm.at[eidb], cvc, sem).wait()
        pltpu.async_copy(z_hbm.at[svc], rows, sem).wait()

        def r16loop(r16, _):
            qsl = pl.ds(r16 * 16, 16)
            loc16 = (pec[qsl] >> 18) & 511
            if gated:
                g16 = ((cvc[qsl] >> 14) & 262143).astype(jnp.float32) \
                    * (1.0 / GSCALE)
            for i in range(16):
                li = loc16[i]
                r = r16 * 16 + i
                for jj in range(D // 16):
                    csl = pl.ds(jj * 16, 16)
                    if gated:
                        acc[li, csl] = acc[li, csl] + rows[r, csl] * g16[i]
                    else:
                        acc[li, csl] = acc[li, csl] + rows[r, csl]
                dr = DEGBASE + li // 16
                oh = jnp.where(iota16 == (li % 16), 1.0, 0.0)
                dsl = pl.ds(0, 16)
                acc[dr, dsl] = acc[dr, dsl] + oh
            return 0
        lax.fori_loop(0, K // 16, r16loop, 0)
        return 0

    lax.fori_loop(0, nch, chunk, 0)

    pltpu.sync_copy(acc, out_hbm.at[pl.ds(w * AROWS, AROWS)])


def _sc_consume(z, queue, cv, gated):
    mesh = plsc.VectorSubcoreMesh(core_axis_name="c", subcore_axis_name="s",
                                  num_cores=NC, num_subcores=NS)
    scratch = [
        pltpu.VMEM((AROWS, D), jnp.float32),
        pltpu.VMEM((16,), jnp.int32),
        pltpu.VMEM((K,), jnp.int32),
        pltpu.VMEM((K,), jnp.int32),
        pltpu.VMEM((K,), jnp.int32),
        pltpu.VMEM((K,), jnp.int32),
        pltpu.VMEM((K, D), jnp.float32),
        pltpu.SemaphoreType.DMA,
    ]
    body = _sc_consume_gated if gated else _sc_consume_plain
    out_type = jax.ShapeDtypeStruct((NW * AROWS, D), jnp.float32)
    fn = pl.kernel(body, out_type=out_type, mesh=mesh, scratch_types=scratch)
    out = fn(z, queue, cv) if gated else fn(z, queue)

    out3 = out.reshape(NW, AROWS, D)
    agg = out3[:, :RPT, :].reshape(NPAD, D)[:N]
    deg = out3[:, DEGBASE:DEGBASE + 20, :16].reshape(NPAD, 1)[:N]
    return agg, deg


def kernel(x_leg, x_bill, edge_attr_vote, W_in_leg, b_in_leg, g_in_leg,
           be_in_leg, W_in_bill, b_in_bill, g_in_bill, be_in_bill,
           W_src_vote, W_dst_vote, b_dst_vote, g_ln_vote, b_ln_vote,
           W_src_rev, W_dst_rev, b_dst_rev, g_ln_rev, b_ln_rev,
           Wg1, bg1, Wg2, bg2, g_res_leg, b_res_leg, g_res_bill, b_res_bill,
           edge_index_vote, edge_index_rev):
    sv = edge_index_vote[0]
    dvv = edge_index_vote[1]
    sr = edge_index_rev[0]
    dr = edge_index_rev[1]

    h_leg = _in_proj(x_leg, W_in_leg, b_in_leg, g_in_leg, be_in_leg)
    h_bill = _in_proj(x_bill, W_in_bill, b_in_bill, g_in_bill, be_in_bill)

    cvs = _gate(edge_attr_vote, dvv, Wg1, bg1, Wg2, bg2)
    q_vote = _sc_build(sv, dvv)
    q_rev = _sc_build(sr, dr)

    deg_v = None
    deg_r = None
    for li in range(L):
        z_v = _matmul(h_leg, W_src_vote[li])
        z_r = _matmul(h_bill, W_src_rev[li])
        agg_bill, dv_new = _sc_consume(z_v, q_vote, cvs[li], True)
        agg_leg, dr_new = _sc_consume(z_r, q_rev, None, False)
        if deg_v is None:
            deg_v, deg_r = dv_new, dr_new
        h_bill_new = _fuse(agg_bill, deg_v, h_bill, W_dst_vote[li],
                           b_dst_vote[li], g_ln_vote[li], b_ln_vote[li],
                           g_res_bill[li], b_res_bill[li])
        h_leg_new = _fuse(agg_leg, deg_r, h_leg, W_dst_rev[li],
                          b_dst_rev[li], g_ln_rev[li], b_ln_rev[li],
                          g_res_leg[li], b_res_leg[li])
        h_leg, h_bill = h_leg_new, h_bill_new

    return jnp.stack([h_leg, h_bill])


# speedup vs baseline: 1.5507x; 1.5507x over previous
import jax
import jax.numpy as jnp
import numpy as np
from jax import lax
from jax.experimental import pallas as pl
from jax.experimental.pallas import tpu as pltpu
from jax.experimental.pallas import tpu_sc as plsc

N = 10000
E = 160000
D = 256
DE = 16
L = 2
DH = D // 2

NC = 2
NS = 16
NW = NC * NS
RPT = 320
NPAD = NW * RPT
JUNK = RPT
DEGBASE = RPT + 8
AROWS = DEGBASE + 24
K = 64


def _ln(x, g, b):
    mu = jnp.mean(x, axis=-1, keepdims=True)
    v = jnp.mean((x - mu) ** 2, axis=-1, keepdims=True)
    return g * (x - mu) / jnp.sqrt(v + 1e-5) + b


def _gelu(x):
    return 0.5 * x * (1.0 + lax.erf(x * 0.7071067811865476))


def _in_proj_body(x_ref, w_ref, b_ref, g_ref, be_ref, o_ref):
    y = jnp.dot(x_ref[...], w_ref[...], preferred_element_type=jnp.float32)
    y = _gelu(y + b_ref[...])
    o_ref[...] = _ln(y, g_ref[...], be_ref[...])


def _matmul_body(x_ref, w_ref, o_ref):
    o_ref[...] = jnp.dot(x_ref[...], w_ref[...],
                         preferred_element_type=jnp.float32)


GSCALE = 262143.0


def _gate_body(ea_ref, dvv_ref, div_ref, A_ref, B_ref, C_ref, bg1_ref,
               w2_ref, bg2_ref, o_ref):
    ea = ea_ref[...]
    ts = ea[:, 0:1]
    ty = ea[:, 1:2]
    dv = div_ref[...]
    s = jnp.sin(ts * dv)
    c = jnp.cos(ty * dv)
    h = jnp.dot(ea, A_ref[...], preferred_element_type=jnp.float32)
    h += jnp.dot(s, B_ref[...], preferred_element_type=jnp.float32)
    h += jnp.dot(c, C_ref[...], preferred_element_type=jnp.float32)
    h = _gelu(h + bg1_ref[...])
    hw = h * w2_ref[...]
    g0 = jax.nn.sigmoid(jnp.sum(hw[:, :D], axis=-1) + bg2_ref[0, 0])
    g1 = jax.nn.sigmoid(jnp.sum(hw[:, D:], axis=-1) + bg2_ref[0, 1])
    dvv = dvv_ref[0, :]
    q0 = jnp.round(g0 * GSCALE).astype(jnp.int32)
    q1 = jnp.round(g1 * GSCALE).astype(jnp.int32)
    o_ref[0, :] = dvv | (q0 << 14)
    o_ref[1, :] = dvv | (q1 << 14)


def _fuse_body(agg_ref, deg_ref, h_ref, wd_ref, bd_ref, gl_ref, bl_ref,
               gr_ref, br_ref, o_ref):
    h = h_ref[...]
    deg = jnp.maximum(deg_ref[...][:, 0:1], 1.0)
    out = agg_ref[...] / deg
    out += jnp.dot(h, wd_ref[...], preferred_element_type=jnp.float32)
    out += bd_ref[...]
    msg = _gelu(_ln(out, gl_ref[...], bl_ref[...]))
    o_ref[...] = _ln(msg + h, gr_ref[...], br_ref[...])


_ROWB = 1000


def _in_proj(x, w, b, g, be):
    xs = pl.BlockSpec((_ROWB, D), lambda i: (i, 0))
    wspec = pl.BlockSpec((D, D), lambda i: (0, 0))
    vspec = pl.BlockSpec((1, D), lambda i: (0, 0))
    return pl.pallas_call(
        _in_proj_body,
        grid=(N // _ROWB,),
        in_specs=[xs, wspec, vspec, vspec, vspec],
        out_specs=xs,
        out_shape=jax.ShapeDtypeStruct((N, D), jnp.float32),
    )(x, w, b.reshape(1, D), g.reshape(1, D), be.reshape(1, D))


def _matmul(x, w):
    xs = pl.BlockSpec((_ROWB, D), lambda i: (i, 0))
    wspec = pl.BlockSpec((D, D), lambda i: (0, 0))
    return pl.pallas_call(
        _matmul_body,
        grid=(N // _ROWB,),
        in_specs=[xs, wspec],
        out_specs=xs,
        out_shape=jax.ShapeDtypeStruct((N, D), jnp.float32),
    )(x, w)


_EB = 2048


def _gate(ea, dvv, Wg1, bg1, Wg2, bg2):
    A = jnp.concatenate([Wg1[0, :DE, :], Wg1[1, :DE, :]], axis=1)
    B = jnp.concatenate([Wg1[0, DE + 0::2, :], Wg1[1, DE + 0::2, :]], axis=1)
    C = jnp.concatenate([Wg1[0, DE + 1::2, :], Wg1[1, DE + 1::2, :]], axis=1)
    bg1c = jnp.concatenate([bg1[0], bg1[1]]).reshape(1, 2 * D)
    w2 = jnp.concatenate([Wg2[0, :, 0], Wg2[1, :, 0]]).reshape(1, 2 * D)
    bg2c = bg2.reshape(1, L)
    div = jnp.exp(jnp.arange(DH, dtype=jnp.float32)
                  * (-(np.log(10000.0) / DH))).reshape(1, DH)

    easpec = pl.BlockSpec((_EB, DE), lambda i: (i, 0))
    dvspec = pl.BlockSpec((1, _EB), lambda i: (0, i))
    ospec = pl.BlockSpec((L, _EB), lambda i: (0, i))
    fulls = [
        pl.BlockSpec((1, DH), lambda i: (0, 0)),
        pl.BlockSpec((DE, 2 * D), lambda i: (0, 0)),
        pl.BlockSpec((DH, 2 * D), lambda i: (0, 0)),
        pl.BlockSpec((DH, 2 * D), lambda i: (0, 0)),
        pl.BlockSpec((1, 2 * D), lambda i: (0, 0)),
        pl.BlockSpec((1, 2 * D), lambda i: (0, 0)),
        pl.BlockSpec((1, L), lambda i: (0, 0)),
    ]
    return pl.pallas_call(
        _gate_body,
        grid=(pl.cdiv(E, _EB),),
        in_specs=[easpec, dvspec] + fulls,
        out_specs=ospec,
        out_shape=jax.ShapeDtypeStruct((L, E), jnp.int32),
    )(ea, dvv.reshape(1, E), div, A, B, C, bg1c, w2, bg2c)


def _fuse(agg, deg, h, wd, bd, gl, bl, gr, br):
    xs = pl.BlockSpec((_ROWB, D), lambda i: (i, 0))
    degspec = pl.BlockSpec((_ROWB, 1), lambda i: (i, 0))
    wspec = pl.BlockSpec((D, D), lambda i: (0, 0))
    vspec = pl.BlockSpec((1, D), lambda i: (0, 0))
    return pl.pallas_call(
        _fuse_body,
        grid=(N // _ROWB,),
        in_specs=[xs, degspec, xs, wspec, vspec, vspec, vspec, vspec, vspec],
        out_specs=xs,
        out_shape=jax.ShapeDtypeStruct((N, D), jnp.float32),
    )(agg, deg, h, wd, bd.reshape(1, D), gl.reshape(1, D), bl.reshape(1, D),
      gr.reshape(1, D), br.reshape(1, D))


QW = E + 16
REG = QW + 16
BSUB = 1280
NBSUB = E // BSUB
GPB = 8
NBATCH = BSUB // (16 * GPB)


def _sc_build_body(sv_hbm, dv_hbm, q_hbm,
                   svt, dvt, pad, idxb, peb, sem):
    c = lax.axis_index("c")
    s = lax.axis_index("s")
    w = c * NS + s
    base = w * RPT
    rbase = w * REG
    iota16 = lax.iota(jnp.int32, 16)
    pad[pl.ds(0, 16)] = jnp.zeros((16,), jnp.float32)

    def block(o, q):
        eo = o * BSUB
        pltpu.sync_copy(sv_hbm.at[pl.ds(eo, BSUB)], svt)
        pltpu.sync_copy(dv_hbm.at[pl.ds(eo, BSUB)], dvt)

        copies = []
        for bi in range(NBATCH):
            for t in range(GPB):
                j = bi * GPB + t
                esl = pl.ds(j * 16, 16)
                loc = (dvt[esl] & 16383) - base
                m = (loc >= 0) & (loc < RPT)
                mi = jnp.where(m, jnp.float32(1), jnp.float32(0))
                p = mi
                for k in (1, 2, 4, 8):
                    pad[pl.ds(16, 16)] = p
                    p = p + pad[pl.ds(16 - k, 16)]
                excl = (p - mi).astype(jnp.int32)
                pos = jnp.where(m, q + excl, E)
                b = t * 16
                idxb[bi, pl.ds(b, 16)] = rbase + pos
                peb[bi, pl.ds(b, 16)] = (eo + j * 16 + iota16) | (loc << 18)
                q = q + p[15].astype(jnp.int32)
            copies.append(
                pltpu.async_copy(peb.at[bi], q_hbm.at[idxb.at[bi]], sem))
        for cp in copies:
            cp.wait()
        return q

    q = lax.fori_loop(0, NBSUB, block, jnp.int32(0))

    pad2 = jnp.zeros((16,), jnp.int32) + q
    idxb[0, pl.ds(0, 16)] = pad2
    pltpu.sync_copy(idxb.at[0, pl.ds(0, 16)],
                    q_hbm.at[pl.ds(rbase + QW, 16)])


def _sc_build(sv, dv):
    mesh = plsc.VectorSubcoreMesh(core_axis_name="c", subcore_axis_name="s",
                                  num_cores=NC, num_subcores=NS)
    scratch = [
        pltpu.VMEM((BSUB,), jnp.int32),
        pltpu.VMEM((BSUB,), jnp.int32),
        pltpu.VMEM((48,), jnp.float32),
        pltpu.VMEM((NBATCH, 16 * GPB), jnp.int32),
        pltpu.VMEM((NBATCH, 16 * GPB), jnp.int32),
        pltpu.SemaphoreType.DMA,
    ]
    fn = pl.kernel(_sc_build_body,
                   out_type=jax.ShapeDtypeStruct((NW * REG,), jnp.int32),
                   mesh=mesh, scratch_types=scratch)
    return fn(sv, dv)


def _sc_consume_gated(z_hbm, q_hbm, sv_hbm, cv_hbm, out_hbm,
                      acc, cntv, svc, pec, eidb, cvc, rows, sem):
    _sc_consume_common(z_hbm, q_hbm, sv_hbm, cv_hbm, out_hbm, acc, cntv,
                       svc, pec, eidb, cvc, rows, sem, gated=True)


def _sc_consume_plain(z_hbm, q_hbm, sv_hbm, out_hbm,
                      acc, cntv, svc, pec, eidb, cvc, rows, sem):
    _sc_consume_common(z_hbm, q_hbm, sv_hbm, None, out_hbm, acc, cntv,
                       svc, pec, eidb, cvc, rows, sem, gated=False)


def _sc_consume_common(z_hbm, q_hbm, sv_hbm, cv_hbm, out_hbm,
                       acc, cntv, svc, pec, eidb, cvc, rows, sem, *, gated):
    c = lax.axis_index("c")
    s = lax.axis_index("s")
    w = c * NS + s
    rbase = w * REG
    iota16 = lax.iota(jnp.int32, 16)
    z16 = jnp.zeros((16,), jnp.float32)

    pltpu.sync_copy(q_hbm.at[pl.ds(rbase + QW, 16)], cntv)
    cnt = cntv[pl.ds(0, 16)][0]

    def _zero(i, _):
        for jj in range(D // 16):
            acc[i, pl.ds(jj * 16, 16)] = z16
        return 0
    lax.fori_loop(0, AROWS, _zero, 0)

    nch = (cnt + (K - 1)) // K

    def chunk(i, _):
        qb = i * K
        pltpu.sync_copy(q_hbm.at[pl.ds(rbase + qb, K)], pec)
        for t in range(K // 16):
            tsl = pl.ds(t * 16, 16)
            valid = (qb + t * 16 + iota16) < cnt
            pe = jnp.where(valid, pec[tsl], JUNK << 18)
            pec[tsl] = pe
            eidb[tsl] = pe & 262143
        csv = pltpu.async_copy(sv_hbm.at[eidb], svc, sem)
        if gated:
            ccv = pltpu.async_copy(cv_hbm.at[eidb], cvc, sem)
        csv.wait()
        if gated:
            ccv.wait()
        pltpu.async_copy(z_hbm.at[svc], rows, sem).wait()

        def r16loop(r16, _):
            qsl = pl.ds(r16 * 16, 16)
            loc16 = (pec[qsl] >> 18) & 511
            if gated:
                g16 = ((cvc[qsl] >> 14) & 262143).astype(jnp.float32) \
                    * (1.0 / GSCALE)
            for i in range(16):
                li = loc16[i]
                r = r16 * 16 + i
                for jj in range(D // 16):
                    csl = pl.ds(jj * 16, 16)
                    if gated:
                        acc[li, csl] = acc[li, csl] + rows[r, csl] * g16[i]
                    else:
                        acc[li, csl] = acc[li, csl] + rows[r, csl]
                dr = DEGBASE + li // 16
                oh = jnp.where(iota16 == (li % 16), 1.0, 0.0)
                dsl = pl.ds(0, 16)
                acc[dr, dsl] = acc[dr, dsl] + oh
            return 0
        lax.fori_loop(0, K // 16, r16loop, 0)
        return 0

    lax.fori_loop(0, nch, chunk, 0)

    pltpu.sync_copy(acc, out_hbm.at[pl.ds(w * AROWS, AROWS)])


def _sc_consume(z, queue, sv, cv, gated):
    mesh = plsc.VectorSubcoreMesh(core_axis_name="c", subcore_axis_name="s",
                                  num_cores=NC, num_subcores=NS)
    scratch = [
        pltpu.VMEM((AROWS, D), jnp.float32),
        pltpu.VMEM((16,), jnp.int32),
        pltpu.VMEM((K,), jnp.int32),
        pltpu.VMEM((K,), jnp.int32),
        pltpu.VMEM((K,), jnp.int32),
        pltpu.VMEM((K,), jnp.int32),
        pltpu.VMEM((K, D), jnp.float32),
        pltpu.SemaphoreType.DMA,
    ]
    body = _sc_consume_gated if gated else _sc_consume_plain
    out_type = jax.ShapeDtypeStruct((NW * AROWS, D), jnp.float32)
    fn = pl.kernel(body, out_type=out_type, mesh=mesh, scratch_types=scratch)
    out = fn(z, queue, sv, cv) if gated else fn(z, queue, sv)

    out3 = out.reshape(NW, AROWS, D)
    agg = out3[:, :RPT, :].reshape(NPAD, D)[:N]
    deg = out3[:, DEGBASE:DEGBASE + 20, :16].reshape(NPAD, 1)[:N]
    return agg, deg


def kernel(x_leg, x_bill, edge_attr_vote, W_in_leg, b_in_leg, g_in_leg,
           be_in_leg, W_in_bill, b_in_bill, g_in_bill, be_in_bill,
           W_src_vote, W_dst_vote, b_dst_vote, g_ln_vote, b_ln_vote,
           W_src_rev, W_dst_rev, b_dst_rev, g_ln_rev, b_ln_rev,
           Wg1, bg1, Wg2, bg2, g_res_leg, b_res_leg, g_res_bill, b_res_bill,
           edge_index_vote, edge_index_rev):
    sv = edge_index_vote[0]
    dvv = edge_index_vote[1]
    sr = edge_index_rev[0]
    dr = edge_index_rev[1]

    h_leg = _in_proj(x_leg, W_in_leg, b_in_leg, g_in_leg, be_in_leg)
    h_bill = _in_proj(x_bill, W_in_bill, b_in_bill, g_in_bill, be_in_bill)

    cvs = _gate(edge_attr_vote, dvv, Wg1, bg1, Wg2, bg2)
    q_vote = _sc_build(sv, dvv)
    q_rev = _sc_build(sr, dr)

    deg_v = None
    deg_r = None
    for li in range(L):
        z_v = _matmul(h_leg, W_src_vote[li])
        z_r = _matmul(h_bill, W_src_rev[li])
        agg_bill, dv_new = _sc_consume(z_v, q_vote, sv, cvs[li], True)
        agg_leg, dr_new = _sc_consume(z_r, q_rev, sr, None, False)
        if deg_v is None:
            deg_v, deg_r = dv_new, dr_new
        h_bill_new = _fuse(agg_bill, deg_v, h_bill, W_dst_vote[li],
                           b_dst_vote[li], g_ln_vote[li], b_ln_vote[li],
                           g_res_bill[li], b_res_bill[li])
        h_leg_new = _fuse(agg_leg, deg_r, h_leg, W_dst_rev[li],
                          b_dst_rev[li], g_ln_rev[li], b_ln_rev[li],
                          g_res_leg[li], b_res_leg[li])
        h_leg, h_bill = h_leg_new, h_bill_new

    return jnp.stack([h_leg, h_bill])


# speedup vs baseline: 33.9886x; 21.9186x over previous
import jax
import jax.numpy as jnp
import numpy as np
from jax import lax
from jax.experimental import pallas as pl
from jax.experimental.pallas import tpu as pltpu
from jax.experimental.pallas import tpu_sc as plsc

N = 10000
E = 160000
D = 256
DE = 16
L = 2
DH = D // 2

NC = 2
NS = 16
NW = NC * NS
RPT = 320
NPAD = NW * RPT
JUNK = RPT
DEGBASE = RPT + 8
AROWS = DEGBASE + 24
K = 64


def _ln(x, g, b):
    mu = jnp.mean(x, axis=-1, keepdims=True)
    v = jnp.mean((x - mu) ** 2, axis=-1, keepdims=True)
    return g * (x - mu) / jnp.sqrt(v + 1e-5) + b


def _gelu(x):
    return 0.5 * x * (1.0 + lax.erf(x * 0.7071067811865476))


def _in_proj_body(x_ref, w_ref, b_ref, g_ref, be_ref, o_ref):
    y = jnp.dot(x_ref[...], w_ref[...], preferred_element_type=jnp.float32)
    y = _gelu(y + b_ref[...])
    o_ref[...] = _ln(y, g_ref[...], be_ref[...])


def _matmul_body(x_ref, w_ref, o_ref):
    o_ref[...] = jnp.dot(x_ref[...], w_ref[...],
                         preferred_element_type=jnp.float32)


GSCALE = 262143.0


def _gate_body(ea_ref, dvv_ref, div_ref, A_ref, B_ref, C_ref, bg1_ref,
               w2_ref, bg2_ref, o_ref):
    ea = ea_ref[...]
    ts = ea[:, 0:1]
    ty = ea[:, 1:2]
    dv = div_ref[...]
    s = jnp.sin(ts * dv)
    c = jnp.cos(ty * dv)
    h = jnp.dot(ea, A_ref[...], preferred_element_type=jnp.float32)
    h += jnp.dot(s, B_ref[...], preferred_element_type=jnp.float32)
    h += jnp.dot(c, C_ref[...], preferred_element_type=jnp.float32)
    h = _gelu(h + bg1_ref[...])
    hw = h * w2_ref[...]
    g0 = jax.nn.sigmoid(jnp.sum(hw[:, :D], axis=-1) + bg2_ref[0, 0])
    g1 = jax.nn.sigmoid(jnp.sum(hw[:, D:], axis=-1) + bg2_ref[0, 1])
    dvv = dvv_ref[0, :]
    q0 = jnp.round(g0 * GSCALE).astype(jnp.int32)
    q1 = jnp.round(g1 * GSCALE).astype(jnp.int32)
    o_ref[0, :] = dvv | (q0 << 14)
    o_ref[1, :] = dvv | (q1 << 14)


def _fuse_body(agg_ref, deg_ref, h_ref, wd_ref, bd_ref, gl_ref, bl_ref,
               gr_ref, br_ref, o_ref):
    h = h_ref[...]
    deg = jnp.maximum(deg_ref[...][:, 0:1], 1.0)
    out = agg_ref[...] / deg
    out += jnp.dot(h, wd_ref[...], preferred_element_type=jnp.float32)
    out += bd_ref[...]
    msg = _gelu(_ln(out, gl_ref[...], bl_ref[...]))
    o_ref[...] = _ln(msg + h, gr_ref[...], br_ref[...])


_ROWB = 1000


def _in_proj(x, w, b, g, be):
    xs = pl.BlockSpec((_ROWB, D), lambda i: (i, 0))
    wspec = pl.BlockSpec((D, D), lambda i: (0, 0))
    vspec = pl.BlockSpec((1, D), lambda i: (0, 0))
    return pl.pallas_call(
        _in_proj_body,
        grid=(N // _ROWB,),
        in_specs=[xs, wspec, vspec, vspec, vspec],
        out_specs=xs,
        out_shape=jax.ShapeDtypeStruct((N, D), jnp.float32),
    )(x, w, b.reshape(1, D), g.reshape(1, D), be.reshape(1, D))


def _matmul(x, w):
    xs = pl.BlockSpec((_ROWB, D), lambda i: (i, 0))
    wspec = pl.BlockSpec((D, D), lambda i: (0, 0))
    return pl.pallas_call(
        _matmul_body,
        grid=(N // _ROWB,),
        in_specs=[xs, wspec],
        out_specs=xs,
        out_shape=jax.ShapeDtypeStruct((N, D), jnp.float32),
    )(x, w)


_EB = 2048


def _gate(ea, dvv, Wg1, bg1, Wg2, bg2):
    A = jnp.concatenate([Wg1[0, :DE, :], Wg1[1, :DE, :]], axis=1)
    B = jnp.concatenate([Wg1[0, DE + 0::2, :], Wg1[1, DE + 0::2, :]], axis=1)
    C = jnp.concatenate([Wg1[0, DE + 1::2, :], Wg1[1, DE + 1::2, :]], axis=1)
    bg1c = jnp.concatenate([bg1[0], bg1[1]]).reshape(1, 2 * D)
    w2 = jnp.concatenate([Wg2[0, :, 0], Wg2[1, :, 0]]).reshape(1, 2 * D)
    bg2c = bg2.reshape(1, L)
    div = jnp.exp(jnp.arange(DH, dtype=jnp.float32)
                  * (-(np.log(10000.0) / DH))).reshape(1, DH)

    easpec = pl.BlockSpec((_EB, DE), lambda i: (i, 0))
    dvspec = pl.BlockSpec((1, _EB), lambda i: (0, i))
    ospec = pl.BlockSpec((L, _EB), lambda i: (0, i))
    fulls = [
        pl.BlockSpec((1, DH), lambda i: (0, 0)),
        pl.BlockSpec((DE, 2 * D), lambda i: (0, 0)),
        pl.BlockSpec((DH, 2 * D), lambda i: (0, 0)),
        pl.BlockSpec((DH, 2 * D), lambda i: (0, 0)),
        pl.BlockSpec((1, 2 * D), lambda i: (0, 0)),
        pl.BlockSpec((1, 2 * D), lambda i: (0, 0)),
        pl.BlockSpec((1, L), lambda i: (0, 0)),
    ]
    return pl.pallas_call(
        _gate_body,
        grid=(pl.cdiv(E, _EB),),
        in_specs=[easpec, dvspec] + fulls,
        out_specs=ospec,
        out_shape=jax.ShapeDtypeStruct((L, E), jnp.int32),
    )(ea, dvv.reshape(1, E), div, A, B, C, bg1c, w2, bg2c)


def _fuse(agg, deg, h, wd, bd, gl, bl, gr, br):
    xs = pl.BlockSpec((_ROWB, D), lambda i: (i, 0))
    degspec = pl.BlockSpec((_ROWB, 1), lambda i: (i, 0))
    wspec = pl.BlockSpec((D, D), lambda i: (0, 0))
    vspec = pl.BlockSpec((1, D), lambda i: (0, 0))
    return pl.pallas_call(
        _fuse_body,
        grid=(N // _ROWB,),
        in_specs=[xs, degspec, xs, wspec, vspec, vspec, vspec, vspec, vspec],
        out_specs=xs,
        out_shape=jax.ShapeDtypeStruct((N, D), jnp.float32),
    )(agg, deg, h, wd, bd.reshape(1, D), gl.reshape(1, D), bl.reshape(1, D),
      gr.reshape(1, D), br.reshape(1, D))


def _sc_consume_gated(z_hbm, sv_hbm, cv_hbm, meta_hbm, out_hbm,
                      acc, metav, svc, cvc, rows, sem):
    _sc_consume_common(z_hbm, sv_hbm, cv_hbm, meta_hbm, out_hbm,
                       acc, metav, svc, cvc, rows, sem, gated=True)


def _sc_consume_plain(z_hbm, sv_hbm, cv_hbm, meta_hbm, out_hbm,
                      acc, metav, svc, cvc, rows, sem):
    _sc_consume_common(z_hbm, sv_hbm, cv_hbm, meta_hbm, out_hbm,
                       acc, metav, svc, cvc, rows, sem, gated=False)


def _sc_consume_common(z_hbm, sv_hbm, cv_hbm, meta_hbm, out_hbm,
                       acc, metav, svc, cvc, rows, sem, *, gated):
    c = lax.axis_index("c")
    s = lax.axis_index("s")
    w = c * NS + s
    base = w * RPT
    iota16 = lax.iota(jnp.int32, 16)
    z16 = jnp.zeros((16,), jnp.float32)

    pltpu.sync_copy(meta_hbm.at[pl.ds(w * 16, 16)], metav)
    m16 = metav[pl.ds(0, 16)]
    wstart = m16[0]
    nch = m16[1]

    def _zero(i, _):
        for jj in range(D // 16):
            acc[i, pl.ds(jj * 16, 16)] = z16
        return 0
    lax.fori_loop(0, AROWS, _zero, 0)

    junkcv = jnp.zeros((16,), jnp.int32) + (base + JUNK)

    def chunk(i, _):
        eb = pl.multiple_of(wstart + i * K, 8)
        pltpu.sync_copy(sv_hbm.at[pl.ds(eb, K)], svc)
        pltpu.sync_copy(cv_hbm.at[pl.ds(eb, K)], cvc)
        for t in range(K // 16):
            tsl = pl.ds(t * 16, 16)
            cv16 = cvc[tsl]
            loc = (cv16 & 16383) - base
            inr = (loc >= 0) & (loc < RPT)
            svc[tsl] = jnp.where(inr, svc[tsl], 0)
            cvc[tsl] = jnp.where(inr, cv16, junkcv)
        pltpu.async_copy(z_hbm.at[svc], rows, sem).wait()

        def r16loop(r16, _):
            qsl = pl.ds(r16 * 16, 16)
            cv16 = cvc[qsl]
            loc16 = (cv16 & 16383) - base
            if gated:
                g16 = ((cv16 >> 14) & 262143).astype(jnp.float32) \
                    * (1.0 / GSCALE)
            for i in range(16):
                li = loc16[i]
                r = r16 * 16 + i
                for jj in range(D // 16):
                    csl = pl.ds(jj * 16, 16)
                    if gated:
                        acc[li, csl] = acc[li, csl] + rows[r, csl] * g16[i]
                    else:
                        acc[li, csl] = acc[li, csl] + rows[r, csl]
                dr = DEGBASE + li // 16
                oh = jnp.where(iota16 == (li % 16), 1.0, 0.0)
                dsl = pl.ds(0, 16)
                acc[dr, dsl] = acc[dr, dsl] + oh
            return 0
        lax.fori_loop(0, K // 16, r16loop, 0)
        return 0

    lax.fori_loop(0, nch, chunk, 0)

    pltpu.sync_copy(acc, out_hbm.at[pl.ds(w * AROWS, AROWS)])


def _sc_consume(z, sv_s, cv_s, meta, gated):
    mesh = plsc.VectorSubcoreMesh(core_axis_name="c", subcore_axis_name="s",
                                  num_cores=NC, num_subcores=NS)
    scratch = [
        pltpu.VMEM((AROWS, D), jnp.float32),
        pltpu.VMEM((16,), jnp.int32),
        pltpu.VMEM((K,), jnp.int32),
        pltpu.VMEM((K,), jnp.int32),
        pltpu.VMEM((K, D), jnp.float32),
        pltpu.SemaphoreType.DMA,
    ]
    body = _sc_consume_gated if gated else _sc_consume_plain
    out_type = jax.ShapeDtypeStruct((NW * AROWS, D), jnp.float32)
    fn = pl.kernel(body, out_type=out_type, mesh=mesh, scratch_types=scratch)
    out = fn(z, sv_s, cv_s, meta)

    out3 = out.reshape(NW, AROWS, D)
    agg = out3[:, :RPT, :].reshape(NPAD, D)[:N]
    deg = out3[:, DEGBASE:DEGBASE + 20, :16].reshape(NPAD, 1)[:N]
    return agg, deg


def _edge_meta(dv_sorted):
    bounds = jnp.arange(NW + 1, dtype=jnp.int32) * RPT
    off = jnp.searchsorted(dv_sorted, bounds).astype(jnp.int32)
    start = off[:NW]
    end = off[1:]
    wstart = (start // K) * K
    nch = (end - wstart + (K - 1)) // K
    meta = jnp.zeros((NW, 16), jnp.int32)
    meta = meta.at[:, 0].set(wstart).at[:, 1].set(nch)
    return meta.reshape(NW * 16)


def kernel(x_leg, x_bill, edge_attr_vote, W_in_leg, b_in_leg, g_in_leg,
           be_in_leg, W_in_bill, b_in_bill, g_in_bill, be_in_bill,
           W_src_vote, W_dst_vote, b_dst_vote, g_ln_vote, b_ln_vote,
           W_src_rev, W_dst_rev, b_dst_rev, g_ln_rev, b_ln_rev,
           Wg1, bg1, Wg2, bg2, g_res_leg, b_res_leg, g_res_bill, b_res_bill,
           edge_index_vote, edge_index_rev):
    sv = edge_index_vote[0]
    dvv = edge_index_vote[1]
    sr = edge_index_rev[0]
    dr = edge_index_rev[1]

    h_leg = _in_proj(x_leg, W_in_leg, b_in_leg, g_in_leg, be_in_leg)
    h_bill = _in_proj(x_bill, W_in_bill, b_in_bill, g_in_bill, be_in_bill)

    cvs = _gate(edge_attr_vote, dvv, Wg1, bg1, Wg2, bg2)

    perm_v = jnp.argsort(dvv)
    sv_s = jnp.pad(sv[perm_v], (0, K))
    cvs_s = jnp.pad(cvs[:, perm_v], ((0, 0), (0, K)),
                    constant_values=16383)
    meta_v = _edge_meta(dvv[perm_v])
    perm_r = jnp.argsort(dr)
    sr_s = jnp.pad(sr[perm_r], (0, K))
    dr_s = jnp.pad(dr[perm_r], (0, K), constant_values=16383)
    meta_r = _edge_meta(dr[perm_r])

    deg_v = None
    deg_r = None
    for li in range(L):
        z_v = _matmul(h_leg, W_src_vote[li])
        z_r = _matmul(h_bill, W_src_rev[li])
        agg_bill, dv_new = _sc_consume(z_v, sv_s, cvs_s[li], meta_v, True)
        agg_leg, dr_new = _sc_consume(z_r, sr_s, dr_s, meta_r, False)
        if deg_v is None:
            deg_v, deg_r = dv_new, dr_new
        h_bill_new = _fuse(agg_bill, deg_v, h_bill, W_dst_vote[li],
                           b_dst_vote[li], g_ln_vote[li], b_ln_vote[li],
                           g_res_bill[li], b_res_bill[li])
        h_leg_new = _fuse(agg_leg, deg_r, h_leg, W_dst_rev[li],
                          b_dst_rev[li], g_ln_rev[li], b_ln_rev[li],
                          g_res_leg[li], b_res_leg[li])
        h_leg, h_bill = h_leg_new, h_bill_new

    return jnp.stack([h_leg, h_bill])
